# Initial kernel scaffold; baseline (speedup 1.0000x reference)
import jax
import jax.numpy as jnp
from jax.experimental import pallas as pl


def kernel(x, edge_index, W1, a1_src, a1_dst, W2, a2_src, a2_dst):
    raise NotImplementedError("write your pallas kernel here")


# speedup vs baseline: 42.3941x; 42.3941x over previous
import functools

import jax
import jax.numpy as jnp
from jax import lax
from jax.experimental import pallas as pl
from jax.experimental.pallas import tpu as pltpu
from jax.experimental.pallas import tpu_sc as plsc

N = 50000
E = 1600000
KP = 1536
NP = 50176
HALF = NP // 2
STRIPE = HALF // 16
EPT = E // 32
PCH = 2000
REG = 50048
ACH = 64
AW = 72

_MESH = dict(core_axis_name="c", subcore_axis_name="s", num_cores=2,
             num_subcores=16)


def _mm1_body(x_ref, w_ref, y_ref, ad_ref, ms_ref, md_ref):
    i = pl.program_id(0)
    y = jnp.dot(x_ref[...], w_ref[...], preferred_element_type=jnp.float32)
    y_ref[...] = y
    ad_ref[...] = y[:, 72:80]
    bs = jnp.max(y[:, 64:72], axis=0, keepdims=True)
    bd = jnp.max(y[:, 72:80], axis=0, keepdims=True)

    @pl.when(i == 0)
    def _():
        ms_ref[...] = jnp.full((1, 8), -jnp.inf, jnp.float32)
        md_ref[...] = jnp.full((1, 8), -jnp.inf, jnp.float32)

    ms_ref[...] = jnp.maximum(ms_ref[...], bs)
    md_ref[...] = jnp.maximum(md_ref[...], bd)


def _layer1_dense(xp, w1e):
    return pl.pallas_call(
        _mm1_body,
        grid=(NP // 256,),
        in_specs=[pl.BlockSpec((256, KP), lambda i: (i, 0)),
                  pl.BlockSpec((KP, 80), lambda i: (0, 0))],
        out_specs=[pl.BlockSpec((256, 80), lambda i: (i, 0)),
                   pl.BlockSpec((256, 8), lambda i: (i, 0)),
                   pl.BlockSpec((1, 8), lambda i: (0, 0)),
                   pl.BlockSpec((1, 8), lambda i: (0, 0))],
        out_shape=[jax.ShapeDtypeStruct((NP, 80), jnp.float32),
                   jax.ShapeDtypeStruct((NP, 8), jnp.float32),
                   jax.ShapeDtypeStruct((1, 8), jnp.float32),
                   jax.ShapeDtypeStruct((1, 8), jnp.float32)],
    )(xp, w1e)


def _mm2_body(h_ref, w_ref, a2s_ref, a2d_ref, ts_ref, td_ref, ms_ref, md_ref):
    i = pl.program_id(0)
    h2 = jnp.dot(h_ref[...], w_ref[...], preferred_element_type=jnp.float32)
    as2 = jnp.sum(h2 * a2s_ref[...], axis=1, keepdims=True)
    ad2 = jnp.sum(h2 * a2d_ref[...], axis=1, keepdims=True)
    lane = lax.broadcasted_iota(jnp.int32, (512, 8), 1)
    ts_ref[...] = jnp.where(lane < 7, h2, jnp.broadcast_to(as2, (512, 8)))
    td_ref[...] = jnp.broadcast_to(ad2, (512, 8))
    bs = jnp.max(jnp.broadcast_to(as2, (512, 8)), axis=0, keepdims=True)
    bd = jnp.max(jnp.broadcast_to(ad2, (512, 8)), axis=0, keepdims=True)

    @pl.when(i == 0)
    def _():
        ms_ref[...] = jnp.full((1, 8), -jnp.inf, jnp.float32)
        md_ref[...] = jnp.full((1, 8), -jnp.inf, jnp.float32)

    ms_ref[...] = jnp.maximum(ms_ref[...], bs)
    md_ref[...] = jnp.maximum(md_ref[...], bd)


def _layer2_dense(h1f, w2p, a2s, a2d):
    return pl.pallas_call(
        _mm2_body,
        grid=(NP // 512,),
        in_specs=[pl.BlockSpec((512, 64), lambda i: (i, 0)),
                  pl.BlockSpec((64, 8), lambda i: (0, 0)),
                  pl.BlockSpec((1, 8), lambda i: (0, 0)),
                  pl.BlockSpec((1, 8), lambda i: (0, 0))],
        out_specs=[pl.BlockSpec((512, 8), lambda i: (i, 0)),
                   pl.BlockSpec((512, 8), lambda i: (i, 0)),
                   pl.BlockSpec((1, 8), lambda i: (0, 0)),
                   pl.BlockSpec((1, 8), lambda i: (0, 0))],
        out_shape=[jax.ShapeDtypeStruct((NP, 8), jnp.float32),
                   jax.ShapeDtypeStruct((NP, 8), jnp.float32),
                   jax.ShapeDtypeStruct((1, 8), jnp.float32),
                   jax.ShapeDtypeStruct((1, 8), jnp.float32)],
    )(h1f, w2p, a2s, a2d)


def _final_body(a_ref, o_ref):
    a = a_ref[...]
    den = a[:, 7:8] + 1e-16
    o = a[:, :7] / den
    m = jnp.max(o, axis=1, keepdims=True)
    z = o - m
    lse = jnp.log(jnp.sum(jnp.exp(z), axis=1, keepdims=True))
    o_ref[...] = z - lse


def _final(acc2):
    return pl.pallas_call(
        _final_body,
        grid=(NP // 512,),
        in_specs=[pl.BlockSpec((512, 8), lambda i: (i, 0))],
        out_specs=pl.BlockSpec((512, 7), lambda i: (i, 0)),
        out_shape=jax.ShapeDtypeStruct((NP, 7), jnp.float32),
    )(acc2)


def _make_partition():
    mesh = plsc.VectorSubcoreMesh(**_MESH)
    scratch = [
        pltpu.VMEM((PCH,), jnp.int32),
        pltpu.VMEM((PCH,), jnp.int32),
        pltpu.VMEM((REG,), jnp.int32),
        pltpu.VMEM((REG,), jnp.int32),
        pltpu.VMEM((16,), jnp.int32),
        pltpu.SemaphoreType.DMA,
    ]

    @functools.partial(
        pl.kernel,
        out_type=(jax.ShapeDtypeStruct((32, 2, REG), jnp.int32),
                  jax.ShapeDtypeStruct((32, 16), jnp.int32)),
        mesh=mesh, scratch_types=scratch,
        compiler_params=pltpu.CompilerParams(
            needs_layout_passes=False, use_tc_tiling_on_sc=False))
    def part(src_h, dst_h, binned_h, meta_h, sbuf, dbuf, st0, st1, mrow, sem):
        c = lax.axis_index("c")
        s = lax.axis_index("s")
        wid = c * 16 + s
        base = wid * EPT
        i16 = lax.broadcasted_iota(jnp.int32, (16,), 0)

        def chunk(j, carry):
            p0, p1 = carry
            pltpu.sync_copy(src_h.at[pl.ds(base + j * PCH, PCH)], sbuf)
            pltpu.sync_copy(dst_h.at[pl.ds(base + j * PCH, PCH)], dbuf)

            def vec(k, carry2):
                q0, q1 = carry2
                sv = sbuf[pl.ds(k * 16, 16)]
                dv = dbuf[pl.ds(k * 16, 16)]
                packed = jnp.bitwise_or(sv, lax.shift_left(dv, 16))
                m0 = dv < HALF
                c0 = plsc.cumsum(m0.astype(jnp.int32))
                c1 = (i16 + 1) - c0
                plsc.store_scatter(st0, [q0 + c0 - 1], packed, mask=m0)
                plsc.store_scatter(st1, [q1 + c1 - 1], packed,
                                   mask=jnp.logical_not(m0))
                n0 = c0[15]
                return (q0 + n0, q1 + (16 - n0))

            return lax.fori_loop(0, PCH // 16, vec, (p0, p1))

        cnt0, cnt1 = lax.fori_loop(0, EPT // PCH, chunk, (0, 0))
        pltpu.sync_copy(st0, binned_h.at[wid, 0])
        pltpu.sync_copy(st1, binned_h.at[wid, 1])
        mrow[...] = jnp.where(i16 == 0, cnt0, jnp.where(i16 == 1, cnt1, 0))
        pltpu.sync_copy(mrow, meta_h.at[wid])

    return part


def _make_agg1():
    mesh = plsc.VectorSubcoreMesh(**_MESH)
    scratch = [
        pltpu.VMEM((ACH,), jnp.int32),
        pltpu.VMEM((ACH,), jnp.int32),
        pltpu.VMEM((ACH,), jnp.int32),
        pltpu.VMEM((ACH,), jnp.int32),
        pltpu.VMEM((ACH,), jnp.float32),
        pltpu.VMEM((ACH, 80), jnp.float32),
        pltpu.VMEM((ACH, 8), jnp.float32),
        pltpu.VMEM((ACH * 8,), jnp.float32),
        pltpu.VMEM((ACH, AW), jnp.float32),
        pltpu.VMEM((16,), jnp.float32),
        pltpu.VMEM((32, 16), jnp.int32),
        pltpu.VMEM((32, AW), jnp.float32),
        pltpu.VMEM((32, 64), jnp.float32),
        pltpu.VMEM_SHARED((HALF, AW), jnp.float32),
        pltpu.SemaphoreType.DMA,
    ]

    @functools.partial(
        pl.kernel,
        out_type=jax.ShapeDtypeStruct((NP, 64), jnp.float32),
        mesh=mesh, scratch_types=scratch,
        compiler_params=pltpu.CompilerParams(
            needs_layout_passes=False, use_tc_tiling_on_sc=False))
    def agg1(binned_h, meta_h, hs_h, ad_h, mt_h, out_h,
             ebuf, sloc, gloc, dloc, vmask, rows, adrows, exbuf, msg,
             mvec, metab, accv, fin, acc, sem):
        c = lax.axis_index("c")
        s = lax.axis_index("s")
        i16 = lax.broadcasted_iota(jnp.int32, (16,), 0)
        hi = lax.shift_right_logical(i16, 3)
        lo7 = jnp.bitwise_and(i16, 7)
        lane8 = i16 < 8
        z16 = jnp.zeros((16,), jnp.float32)

        pltpu.sync_copy(mt_h, mvec)
        pltpu.sync_copy(meta_h, metab)

        def zrow(p, _):
            for sub in range(2):
                for k in range(4):
                    msg[2 * p + sub, pl.ds(k * 16, 16)] = z16
            plsc.store_scatter(msg, [2 * p + hi, 64 + lo7], z16)
            return 0
        lax.fori_loop(0, ACH // 2, zrow, 0)

        sb = s * STRIPE
        for r in range(STRIPE // ACH):
            pltpu.sync_copy(msg, acc.at[pl.ds(sb + r * ACH, ACH)])
        pltpu.sync_copy(msg.at[pl.ds(0, STRIPE % ACH)],
                        acc.at[pl.ds(sb + (STRIPE // ACH) * ACH,
                                     STRIPE % ACH)])
        plsc.subcore_barrier()

        mv = mvec[...]
        idxk = [[(sub * 8 + 2 * k) + hi for k in range(4)]
                for sub in range(2)]
        idx4 = [sub * 8 + lo7 for sub in range(2)]

        for t in (s, s + 16):
            row = metab[t, pl.ds(0, 16)]
            cnt = lax.reduce_max(jnp.where(i16 == c, row, 0), (0,))
            nch = (cnt + ACH - 1) // ACH

            def chunk(j, _):
                pltpu.sync_copy(binned_h.at[t, c, pl.ds(j * ACH, ACH)], ebuf)

                def up(k, _2):
                    e = ebuf[pl.ds(k * 16, 16)]
                    sv = jnp.bitwise_and(e, 0xFFFF)
                    dv = lax.shift_right_logical(e, 16)
                    val = i16 < (cnt - j * ACH - k * 16)
                    sloc[pl.ds(k * 16, 16)] = jnp.where(val, sv, 0)
                    gloc[pl.ds(k * 16, 16)] = jnp.where(val, dv, 0)
                    dloc[pl.ds(k * 16, 16)] = jnp.where(val, dv - c * HALF, 0)
                    vmask[pl.ds(k * 16, 16)] = jnp.where(val, 1.0, 0.0)
                    return 0
                lax.fori_loop(0, ACH // 16, up, 0)

                pltpu.sync_copy(hs_h.at[sloc], rows)
                pltpu.sync_copy(ad_h.at[gloc], adrows)

                def exb(p, _2):
                    ri = hi + 2 * p
                    asv = plsc.load_gather(rows, [ri, 64 + lo7])
                    adv = plsc.load_gather(adrows, [ri, lo7])
                    vm = plsc.load_gather(vmask, [ri])
                    zz = asv + adv
                    lr = jnp.maximum(zz, 0.2 * zz) - mv
                    exbuf[pl.ds(p * 16, 16)] = jnp.exp(lr) * vm
                    return 0
                lax.fori_loop(0, ACH // 2, exb, 0)

                def mb(p, _2):
                    for sub in range(2):
                        a = 2 * p + sub
                        for k in range(4):
                            hv = rows[a, pl.ds(k * 16, 16)]
                            pat = plsc.load_gather(
                                exbuf, [p * 16 + idxk[sub][k]])
                            msg[a, pl.ds(k * 16, 16)] = hv * pat
                    ep = exbuf[pl.ds(p * 16, 16)]
                    plsc.store_scatter(msg, [2 * p + hi, 64 + lo7], ep)
                    return 0
                lax.fori_loop(0, ACH // 2, mb, 0)

                pltpu.sync_copy(msg, acc.at[dloc], add=True)
                return 0
            lax.fori_loop(0, nch, chunk, 0)

        plsc.subcore_barrier()

        def finb(r, _):
            rb = sb + r * 32
            pltpu.sync_copy(acc.at[pl.ds(rb, 32)], accv)

            def rw(a, _2):
                af = i16 * 0 + a
                for k in range(4):
                    num = accv[a, pl.ds(k * 16, 16)]
                    den = plsc.load_gather(accv, [af, 64 + 2 * k + hi])
                    q = num / (den + 1e-16)
                    ev = jnp.exp(jnp.minimum(q, 0.0)) - 1.0
                    fin[a, pl.ds(k * 16, 16)] = jnp.where(q > 0.0, q, ev)
                return 0
            lax.fori_loop(0, 32, rw, 0)
            pltpu.sync_copy(fin, out_h.at[pl.ds(c * HALF + rb, 32)])
            return 0
        lax.fori_loop(0, STRIPE // 32, finb, 0)

    return agg1


def _make_agg2():
    mesh = plsc.VectorSubcoreMesh(**_MESH)
    scratch = [
        pltpu.VMEM((ACH,), jnp.int32),
        pltpu.VMEM((ACH,), jnp.int32),
        pltpu.VMEM((ACH,), jnp.int32),
        pltpu.VMEM((ACH,), jnp.int32),
        pltpu.VMEM((ACH,), jnp.float32),
        pltpu.VMEM((ACH, 8), jnp.float32),
        pltpu.VMEM((ACH,), jnp.float32),
        pltpu.VMEM((ACH,), jnp.float32),
        pltpu.VMEM((ACH, 8), jnp.float32),
        pltpu.VMEM((16,), jnp.float32),
        pltpu.VMEM((32, 16), jnp.int32),
        pltpu.VMEM_SHARED((HALF, 8), jnp.float32),
        pltpu.SemaphoreType.DMA,
    ]

    @functools.partial(
        pl.kernel,
        out_type=jax.ShapeDtypeStruct((NP, 8), jnp.float32),
        mesh=mesh, scratch_types=scratch,
        compiler_params=pltpu.CompilerParams(
            needs_layout_passes=False, use_tc_tiling_on_sc=False))
    def agg2(binned_h, meta_h, ts_h, adf_h, mt_h, out_h,
             ebuf, sloc, gloc, dloc, vmask, rows2, advals, exb2, msg2,
             m2v, metab, acc2, sem):
        c = lax.axis_index("c")
        s = lax.axis_index("s")
        i16 = lax.broadcasted_iota(jnp.int32, (16,), 0)
        hi = lax.shift_right_logical(i16, 3)
        lo7 = jnp.bitwise_and(i16, 7)
        is7 = lo7 == 7
        z16 = jnp.zeros((16,), jnp.float32)

        pltpu.sync_copy(mt_h, m2v)
        pltpu.sync_copy(meta_h, metab)

        def zp(p, _):
            plsc.store_scatter(msg2, [2 * p + hi, lo7], z16)
            return 0
        lax.fori_loop(0, ACH // 2, zp, 0)

        sb = s * STRIPE
        for r in range(STRIPE // ACH):
            pltpu.sync_copy(msg2, acc2.at[pl.ds(sb + r * ACH, ACH)])
        pltpu.sync_copy(msg2.at[pl.ds(0, STRIPE % ACH)],
                        acc2.at[pl.ds(sb + (STRIPE // ACH) * ACH,
                                      STRIPE % ACH)])
        plsc.subcore_barrier()

        mv = m2v[...]

        for t in (s, s + 16):
            row = metab[t, pl.ds(0, 16)]
            cnt = lax.reduce_max(jnp.where(i16 == c, row, 0), (0,))
            nch = (cnt + ACH - 1) // ACH

            def chunk(j, _):
                pltpu.sync_copy(binned_h.at[t, c, pl.ds(j * ACH, ACH)], ebuf)

                def up(k, _2):
                    e = ebuf[pl.ds(k * 16, 16)]
                    sv = jnp.bitwise_and(e, 0xFFFF)
                    dv = lax.shift_right_logical(e, 16)
                    val = i16 < (cnt - j * ACH - k * 16)
                    sloc[pl.ds(k * 16, 16)] = jnp.where(val, sv, 0)
                    gloc[pl.ds(k * 16, 16)] = jnp.where(val, dv, 0)
                    dloc[pl.ds(k * 16, 16)] = jnp.where(val, dv - c * HALF, 0)
                    vmask[pl.ds(k * 16, 16)] = jnp.where(val, 1.0, 0.0)
                    return 0
                lax.fori_loop(0, ACH // 16, up, 0)

                pltpu.sync_copy(ts_h.at[sloc], rows2)
                pltpu.sync_copy(adf_h.at[gloc], advals)

                def exq(q, _2):
                    ri = i16 + q * 16
                    asv = plsc.load_gather(rows2, [ri, i16 * 0 + 7])
                    adv = advals[pl.ds(q * 16, 16)]
                    vm = vmask[pl.ds(q * 16, 16)]
                    zz = asv + adv
                    lr = jnp.maximum(zz, 0.2 * zz) - mv
                    exb2[pl.ds(q * 16, 16)] = jnp.exp(lr) * vm
                    return 0
                lax.fori_loop(0, ACH // 16, exq, 0)

                def mp(p, _2):
                    ri = 2 * p + hi
                    rv = plsc.load_gather(rows2, [ri, lo7])
                    ep = plsc.load_gather(exb2, [ri])
                    m = jnp.where(is7, 1.0, rv) * ep
                    plsc.store_scatter(msg2, [ri, lo7], m)
                    return 0
                lax.fori_loop(0, ACH // 2, mp, 0)

                pltpu.sync_copy(msg2, acc2.at[dloc], add=True)
                return 0
            lax.fori_loop(0, nch, chunk, 0)

        plsc.subcore_barrier()
        pltpu.sync_copy(acc2.at[pl.ds(sb, STRIPE)],
                        out_h.at[pl.ds(c * HALF + sb, STRIPE)])

    return agg2


_partition = _make_partition()
_agg1 = _make_agg1()
_agg2 = _make_agg2()


def kernel(x, edge_index, W1, a1_src, a1_dst, W2, a2_src, a2_dst):
    xp = jnp.pad(x, ((0, NP - N), (0, KP - x.shape[1])))
    i64 = jnp.arange(64)
    asrc = jnp.zeros((64, 8), jnp.float32).at[i64, i64 // 8].set(
        a1_src.reshape(-1))
    adst = jnp.zeros((64, 8), jnp.float32).at[i64, i64 // 8].set(
        a1_dst.reshape(-1))
    w1e = jnp.concatenate([W1, W1 @ asrc, W1 @ adst], axis=1)
    w1e = jnp.pad(w1e, ((0, KP - W1.shape[0]), (0, 0)))

    hs1, ad1, ms1, md1 = _layer1_dense(xp, w1e)
    m1 = jnp.tile((ms1 + md1).reshape(8), 2)

    src = edge_index[0]
    dst = edge_index[1]
    binned, meta = _partition(src, dst)

    h1f = _agg1(binned, meta, hs1, ad1, m1)

    w2p = jnp.pad(W2, ((0, 0), (0, 1)))
    a2s = jnp.pad(a2_src, ((0, 0), (0, 1)))
    a2d = jnp.pad(a2_dst, ((0, 0), (0, 1)))
    ts2, td2, ms2, md2 = _layer2_dense(h1f, w2p, a2s, a2d)
    m2 = jnp.full((16,), (ms2[0, 0] + md2[0, 0]), jnp.float32)

    acc2 = _agg2(binned, meta, ts2, td2[:, 0], m2)

    return _final(acc2)[:N]
